# baseline (speedup 1.0000x reference)
import functools

import jax
import jax.numpy as jnp
from jax import lax
from jax.experimental import pallas as pl
from jax.experimental.pallas import tpu as pltpu
from jax.experimental.pallas import tpu_sc as plsc

VOCAB = 1000000
SEQ = 200
DIM = 64
BATCH = 4096

NC = 2
NS = 16
NW = NC * NS
LANES = 16

ROWS = BATCH * SEQ
RPW = ROWS // NW
SUB = 100
SEQS_PER_CHUNK = 2
CHUNK = SEQ * SEQS_PER_CHUNK
SUBS_PER_CHUNK = CHUNK // SUB
NCHUNK = RPW // CHUNK
NSUB = RPW // SUB
SCALE = 8.0
NBUF = 2

_mesh = plsc.VectorSubcoreMesh(core_axis_name="c", subcore_axis_name="s")


@functools.partial(
    pl.kernel,
    out_type=jax.ShapeDtypeStruct((ROWS, DIM), jnp.float32),
    mesh=_mesh,
    compiler_params=pltpu.CompilerParams(use_tc_tiling_on_sc=False),
    scratch_types=[
        pltpu.VMEM((NSUB, SUB), jnp.int32),
        pltpu.VMEM((NBUF, CHUNK, DIM), jnp.float32),
        pltpu.VMEM((SEQ, DIM), jnp.float32),
        pltpu.SemaphoreType.DMA((NBUF,)),
        pltpu.SemaphoreType.DMA((NBUF,)),
    ],
)
def _embed(idx_hbm, tok_hbm, pos_hbm, out_hbm, idx_v, rows_v, pos_v, gsem, osem):
    wid = lax.axis_index("s") * NC + lax.axis_index("c")
    pltpu.sync_copy(idx_hbm.at[wid], idx_v)
    pltpu.sync_copy(pos_hbm, pos_v)
    base = wid * RPW

    def fire_gather(c, b):
        for k in range(SUBS_PER_CHUNK):
            pltpu.async_copy(
                tok_hbm.at[idx_v.at[c * SUBS_PER_CHUNK + k]],
                rows_v.at[b, pl.ds(k * SUB, SUB)],
                gsem.at[b],
            )

    def wait_gather(c, b):
        for k in range(SUBS_PER_CHUNK):
            pltpu.make_async_copy(
                tok_hbm.at[idx_v.at[c * SUBS_PER_CHUNK + k]],
                rows_v.at[b, pl.ds(k * SUB, SUB)],
                gsem.at[b],
            ).wait()

    def fire_store(c, b):
        pltpu.async_copy(
            rows_v.at[b], out_hbm.at[pl.ds(base + c * CHUNK, CHUNK)], osem.at[b]
        )

    def wait_store(c, b):
        pltpu.make_async_copy(
            rows_v.at[b], out_hbm.at[pl.ds(base + c * CHUNK, CHUNK)], osem.at[b]
        ).wait()

    fire_gather(0, 0)

    def chunk_body(c, carry):
        b = lax.rem(c, 2)
        nb = 1 - b
        wait_gather(c, b)

        @pl.when(c >= 1)
        def _():
            wait_store(c - 1, nb)

        @pl.when(c + 1 < NCHUNK)
        def _():
            fire_gather(c + 1, nb)

        def row_body(r, carry2):
            for g in range(DIM // LANES):
                sl = pl.ds(g * LANES, LANES)
                p = pos_v[r, sl]
                for s in range(SEQS_PER_CHUNK):
                    row = s * SEQ + r
                    rows_v[b, row, sl] = rows_v[b, row, sl] * SCALE + p
            return carry2

        lax.fori_loop(0, SEQ, row_body, 0)
        fire_store(c, b)
        return carry

    lax.fori_loop(0, NCHUNK, chunk_body, 0)
    wait_store(NCHUNK - 1, (NCHUNK - 1) % 2)


def kernel(inputs, token_table, position_table):
    idx = inputs.reshape(NW, NSUB, SUB)
    out = _embed(idx, token_table, position_table)
    return out.reshape(BATCH, SEQ, DIM)


# speedup vs baseline: 1.0028x; 1.0028x over previous
import functools

import jax
import jax.numpy as jnp
from jax import lax
from jax.experimental import pallas as pl
from jax.experimental.pallas import tpu as pltpu
from jax.experimental.pallas import tpu_sc as plsc

VOCAB = 1000000
SEQ = 200
DIM = 64
BATCH = 4096

NC = 2
NS = 16
NW = NC * NS
LANES = 16

SPW = BATCH // NW
SEQS_PER_CHUNK = 2
SUBA = 128
SUBB = SEQ - SUBA
NCHUNK = SPW // SEQS_PER_CHUNK
SCALE = 8.0
NBUF = 2

_mesh = plsc.VectorSubcoreMesh(core_axis_name="c", subcore_axis_name="s")


@functools.partial(
    pl.kernel,
    out_type=jax.ShapeDtypeStruct((BATCH, SEQ, DIM), jnp.float32),
    mesh=_mesh,
    compiler_params=pltpu.CompilerParams(use_tc_tiling_on_sc=False),
    scratch_types=[
        pltpu.VMEM((SPW, SUBA), jnp.int32),
        pltpu.VMEM((SPW, SUBB), jnp.int32),
        pltpu.VMEM((NBUF, SEQS_PER_CHUNK, SEQ, DIM), jnp.float32),
        pltpu.VMEM((SEQ, DIM), jnp.float32),
        pltpu.SemaphoreType.DMA((NBUF,)),
        pltpu.SemaphoreType.DMA((NBUF,)),
    ],
)
def _embed(idxa_hbm, idxb_hbm, tok_hbm, pos_hbm, out_hbm,
           idxa_v, idxb_v, rows_v, pos_v, gsem, osem):
    wid = lax.axis_index("s") * NC + lax.axis_index("c")
    seq0 = wid * SPW
    pltpu.sync_copy(idxa_hbm.at[pl.ds(seq0, SPW)], idxa_v)
    pltpu.sync_copy(idxb_hbm.at[pl.ds(seq0, SPW)], idxb_v)
    pltpu.sync_copy(pos_hbm, pos_v)

    def fire_gather(c, b):
        for j in range(SEQS_PER_CHUNK):
            s = SEQS_PER_CHUNK * c + j
            pltpu.async_copy(
                tok_hbm.at[idxa_v.at[s]],
                rows_v.at[b, j, pl.ds(0, SUBA)],
                gsem.at[b],
            )
            pltpu.async_copy(
                tok_hbm.at[idxb_v.at[s]],
                rows_v.at[b, j, pl.ds(SUBA, SUBB)],
                gsem.at[b],
            )

    def wait_gather(c, b):
        for j in range(SEQS_PER_CHUNK):
            s = SEQS_PER_CHUNK * c + j
            pltpu.make_async_copy(
                tok_hbm.at[idxa_v.at[s]],
                rows_v.at[b, j, pl.ds(0, SUBA)],
                gsem.at[b],
            ).wait()
            pltpu.make_async_copy(
                tok_hbm.at[idxb_v.at[s]],
                rows_v.at[b, j, pl.ds(SUBA, SUBB)],
                gsem.at[b],
            ).wait()

    def fire_store(c, b):
        for j in range(SEQS_PER_CHUNK):
            pltpu.async_copy(
                rows_v.at[b, j],
                out_hbm.at[seq0 + SEQS_PER_CHUNK * c + j],
                osem.at[b],
            )

    def wait_store(c, b):
        for j in range(SEQS_PER_CHUNK):
            pltpu.make_async_copy(
                rows_v.at[b, j],
                out_hbm.at[seq0 + SEQS_PER_CHUNK * c + j],
                osem.at[b],
            ).wait()

    fire_gather(0, 0)

    def chunk_body(c, carry):
        b = lax.rem(c, 2)
        nb = 1 - b
        wait_gather(c, b)

        @pl.when(c >= 1)
        def _():
            wait_store(c - 1, nb)

        @pl.when(c + 1 < NCHUNK)
        def _():
            fire_gather(c + 1, nb)

        def row_body(r, carry2):
            for g in range(DIM // LANES):
                sl = pl.ds(g * LANES, LANES)
                p = pos_v[r, sl]
                for s in range(SEQS_PER_CHUNK):
                    rows_v[b, s, r, sl] = rows_v[b, s, r, sl] * SCALE + p
            return carry2

        lax.fori_loop(0, SEQ, row_body, 0)
        fire_store(c, b)
        return carry

    lax.fori_loop(0, NCHUNK, chunk_body, 0)
    wait_store(NCHUNK - 1, (NCHUNK - 1) % 2)


def kernel(inputs, token_table, position_table):
    idxa = inputs[:, :SUBA]
    idxb = inputs[:, SUBA:]
    return _embed(idxa, idxb, token_table, position_table)
